# baseline (speedup 1.0000x reference)
import functools

import jax
import jax.numpy as jnp
from jax import lax
from jax.experimental import pallas as pl
from jax.experimental.pallas import tpu as pltpu
from jax.experimental.pallas import tpu_sc as plsc

N, D, B, MAXDEG = 10000, 128, 1024, 32
S1, S2 = 25, 10
H = 128

NC, NS = 2, 16
NW = NC * NS
NB = B // NW
NSMP = NB * S2
GS = 80
NSTRIP = NSMP // GS
LANES = 16
PACK = 128 // MAXDEG


def _sc_gather_aggregate(features, adj4, batch1):
  mesh = plsc.VectorSubcoreMesh(core_axis_name="c", subcore_axis_name="s",
                                num_cores=NC, num_subcores=NS)

  @functools.partial(
      pl.kernel,
      out_type=[
          jax.ShapeDtypeStruct((B, D), jnp.float32),
          jax.ShapeDtypeStruct((B * S2, D), jnp.float32),
          jax.ShapeDtypeStruct((B * S2, D), jnp.float32),
      ],
      mesh=mesh,
      compiler_params=pltpu.CompilerParams(needs_layout_passes=False),
      scratch_types=[
          pltpu.VMEM((NB,), jnp.int32),
          pltpu.VMEM((NB, 128), jnp.int32),
          pltpu.VMEM((NSMP,), jnp.int32),
          pltpu.VMEM((NB, D), jnp.float32),
          [pltpu.VMEM((GS,), jnp.int32)] * 2,
          [pltpu.VMEM((GS,), jnp.int32)] * 2,
          [pltpu.VMEM((GS, 128), jnp.int32)] * 2,
          [pltpu.VMEM((GS, D), jnp.float32)] * 2,
          [pltpu.VMEM((GS, D), jnp.float32)] * 2,
          pltpu.VMEM((2 * GS, D), jnp.float32),
          pltpu.VMEM((2 * GS, D), jnp.float32),
          pltpu.VMEM((2 * GS,), jnp.int32),
          pltpu.VMEM((2 * GS,), jnp.int32),
          pltpu.SemaphoreType.DMA,
          pltpu.SemaphoreType.DMA,
          [pltpu.SemaphoreType.DMA] * 2,
          [pltpu.SemaphoreType.DMA] * 2,
          [pltpu.SemaphoreType.DMA] * 2,
          [pltpu.SemaphoreType.DMA] * 2,
      ],
  )
  def body(features_hbm, adj4_hbm, batch1_hbm, h0_out, h1_out, ns1_out,
           bidx_v, adj0_v, samp1_v, h0_v, sidx, soff, adj1, h1, ns1,
           colA_v, colB_v, idxA_v, idxB_v, semA, semB,
           semJ, semG, semH, semN):
    wid = lax.axis_index("c") * NS + lax.axis_index("s")
    base_b = wid * NB
    lane = lax.iota(jnp.int32, LANES)

    pltpu.sync_copy(batch1_hbm.at[pl.ds(base_b, NB)], bidx_v)

    pltpu.sync_copy(features_hbm.at[bidx_v], h0_v)
    pltpu.sync_copy(h0_v, h0_out.at[pl.ds(base_b, NB)])

    for j in range(NB // LANES):
      b = bidx_v[pl.ds(j * LANES, LANES)]
      bidx_v[pl.ds(j * LANES, LANES)] = b // PACK
    pltpu.sync_copy(adj4_hbm.at[bidx_v], adj0_v)

    pltpu.sync_copy(batch1_hbm.at[pl.ds(base_b, NB)], sidx[0].at[pl.ds(0, NB)])
    for j in range(NSMP // LANES):
      p = lane + j * LANES
      row = p // S2
      col = p - row * S2
      boff = (plsc.load_gather(sidx[0], [row]) % PACK) * MAXDEG
      v = plsc.load_gather(adj0_v, [row, boff + col])
      samp1_v[pl.ds(j * LANES, LANES)] = v

    def extract(l, idx_ref, bank, base=0):
      for j in range(GS // LANES):
        rows = lane + j * LANES
        off = soff[bank][pl.ds(j * LANES, LANES)]
        v = plsc.load_gather(adj1[bank], [rows, off + l])
        idx_ref[pl.ds(base + j * LANES, LANES)] = v

    def extract_pair(pp, idx_ref, bank):
      extract(2 * pp, idx_ref, bank, 0)
      extract(2 * pp + 1, idx_ref, bank, GS)

    def fire(idx_ref, col_ref, sem):
      pltpu.async_copy(features_hbm.at[idx_ref], col_ref, sem)

    def wait(idx_ref, col_ref, sem):
      pltpu.make_async_copy(features_hbm.at[idx_ref], col_ref, sem).wait()

    def acc_pair(col_ref, bank, first):
      @pl.loop(0, GS, unroll=4)
      def _a(r):
        for c in range(D // LANES):
          sl = pl.ds(c * LANES, LANES)
          if first:
            ns1[bank][r, sl] = col_ref[r, sl]
          else:
            plsc.addupdate(ns1[bank].at[r, sl], col_ref[r, sl])
          plsc.addupdate(ns1[bank].at[r, sl], col_ref[GS + r, sl])

    def acc_single(col_ref, bank):
      @pl.loop(0, GS, unroll=4)
      def _a(r):
        for c in range(D // LANES):
          sl = pl.ds(c * LANES, LANES)
          plsc.addupdate(ns1[bank].at[r, sl], col_ref[r, sl])

    def prep(t, bank):
      sbase = t * GS
      for j in range(GS // LANES):
        s = samp1_v[pl.ds(sbase + j * LANES, LANES)]
        soff[bank][pl.ds(j * LANES, LANES)] = (s % PACK) * MAXDEG
        sidx[bank][pl.ds(j * LANES, LANES)] = s // PACK

    def fire_strip(t, bank):
      pltpu.async_copy(adj4_hbm.at[sidx[bank]], adj1[bank], semJ[bank])
      pltpu.async_copy(
          features_hbm.at[samp1_v.at[pl.ds(t * GS, GS)]], h1[bank],
          semG[bank])

    def run_strip(t, bank, k, first_pair, last_pair):
      gbase = wid * NSMP + t * GS
      nbank = 1 - bank
      pltpu.make_async_copy(adj4_hbm.at[sidx[bank]], adj1[bank],
                            semJ[bank]).wait()

      if bank == 0:
        @pl.when(jnp.logical_not(first_pair))
        def _():
          pltpu.make_async_copy(h1[1], h1_out.at[pl.ds(0, GS)],
                                semH[1]).wait()
        prep(t + 1, nbank)
        fire_strip(t + 1, nbank)
      else:
        @pl.when(jnp.logical_not(last_pair))
        def _():
          pltpu.make_async_copy(h1[0], h1_out.at[pl.ds(0, GS)],
                                semH[0]).wait()
          prep(t + 1, nbank)
          fire_strip(t + 1, nbank)

      pltpu.make_async_copy(
          features_hbm.at[samp1_v.at[pl.ds(t * GS, GS)]], h1[bank],
          semG[bank]).wait()
      pltpu.async_copy(h1[bank], h1_out.at[pl.ds(gbase, GS)], semH[bank])

      @pl.when(jnp.logical_not(first_pair))
      def _():
        pltpu.make_async_copy(ns1[bank], ns1_out.at[pl.ds(0, GS)],
                              semN[bank]).wait()

      NP = S1 // 2
      extract_pair(0, idxA_v, bank)
      fire(idxA_v, colA_v, semA)
      extract_pair(1, idxB_v, bank)
      fire(idxB_v, colB_v, semB)
      wait(idxA_v, colA_v, semA)
      acc_pair(colA_v, bank, True)
      extract_pair(2, idxA_v, bank)
      fire(idxA_v, colA_v, semA)

      @pl.loop(0, (NP - 4) // 2)
      def _k(kk):
        wait(idxB_v, colB_v, semB)
        acc_pair(colB_v, bank, False)
        extract_pair(2 * kk + 3, idxB_v, bank)
        fire(idxB_v, colB_v, semB)
        wait(idxA_v, colA_v, semA)
        acc_pair(colA_v, bank, False)
        extract_pair(2 * kk + 4, idxA_v, bank)
        fire(idxA_v, colA_v, semA)

      wait(idxB_v, colB_v, semB)
      acc_pair(colB_v, bank, False)
      extract_pair(NP - 1, idxB_v, bank)
      fire(idxB_v, colB_v, semB)
      wait(idxA_v, colA_v, semA)
      acc_pair(colA_v, bank, False)
      extract(S1 - 1, idxA_v, bank, 0)
      pltpu.async_copy(features_hbm.at[idxA_v.at[pl.ds(0, GS)]],
                       colA_v.at[pl.ds(0, GS)], semA)
      wait(idxB_v, colB_v, semB)
      acc_pair(colB_v, bank, False)
      pltpu.make_async_copy(features_hbm.at[idxA_v.at[pl.ds(0, GS)]],
                            colA_v.at[pl.ds(0, GS)], semA).wait()
      acc_single(colA_v, bank)

      pltpu.async_copy(ns1[bank], ns1_out.at[pl.ds(gbase, GS)], semN[bank])

    prep(0, 0)
    fire_strip(0, 0)

    @pl.loop(0, NSTRIP // 2)
    def _pair(k):
      first = k == 0
      last = k == (NSTRIP // 2 - 1)
      run_strip(2 * k, 0, k, first, last)
      run_strip(2 * k + 1, 1, k, first, last)

    for bank in (0, 1):
      pltpu.make_async_copy(h1[bank], h1_out.at[pl.ds(0, GS)],
                            semH[bank]).wait()
      pltpu.make_async_copy(ns1[bank], ns1_out.at[pl.ds(0, GS)],
                            semN[bank]).wait()

  return body(features, adj4, batch1)


def _tc_body(h0_ref, h1_ref, ns1_ref, ws0_ref, wn0_ref, ws1_ref, wn1_ref,
             out_ref):
  f32 = jnp.float32
  h1 = h1_ref[...]
  ns1 = ns1_ref[...] / float(S1)
  ws0 = ws0_ref[...]
  wn0 = wn0_ref[...]
  hs1 = jnp.dot(h1, ws0, preferred_element_type=f32)
  hn1 = jnp.dot(ns1, wn0, preferred_element_type=f32)
  h1c = jnp.maximum(jnp.concatenate([hs1, hn1], axis=1), 0.0)

  bb = h0_ref.shape[0]
  ri = lax.broadcasted_iota(jnp.int32, (bb, bb * S2), 0)
  ci = lax.broadcasted_iota(jnp.int32, (bb, bb * S2), 1)
  pool = jnp.where(ci // S2 == ri, 1.0 / S2, 0.0).astype(f32)

  ns0 = jnp.dot(pool, h1, preferred_element_type=f32)
  h0 = h0_ref[...]
  h0c = jnp.maximum(
      jnp.concatenate([jnp.dot(h0, ws0, preferred_element_type=f32),
                       jnp.dot(ns0, wn0, preferred_element_type=f32)],
                      axis=1), 0.0)
  h1m = jnp.dot(pool, h1c, preferred_element_type=f32)
  out_ref[...] = jnp.concatenate(
      [jnp.dot(h0c, ws1_ref[...], preferred_element_type=f32),
       jnp.dot(h1m, wn1_ref[...], preferred_element_type=f32)], axis=1)


def _tc_aggregate(h0, h1, ns1, W_self_0, W_neigh_0, W_self_1, W_neigh_1):
  Bb = 128
  grid = (B // Bb,)
  return pl.pallas_call(
      _tc_body,
      out_shape=jax.ShapeDtypeStruct((B, 2 * H), jnp.float32),
      grid=grid,
      in_specs=[
          pl.BlockSpec((Bb, D), lambda i: (i, 0)),
          pl.BlockSpec((Bb * S2, D), lambda i: (i, 0)),
          pl.BlockSpec((Bb * S2, D), lambda i: (i, 0)),
          pl.BlockSpec((D, H), lambda i: (0, 0)),
          pl.BlockSpec((D, H), lambda i: (0, 0)),
          pl.BlockSpec((2 * H, H), lambda i: (0, 0)),
          pl.BlockSpec((2 * H, H), lambda i: (0, 0)),
      ],
      out_specs=pl.BlockSpec((Bb, 2 * H), lambda i: (i, 0)),
  )(h0, h1, ns1, W_self_0, W_neigh_0, W_self_1, W_neigh_1)


def kernel(features, adj, batch1, W_self_0, W_neigh_0, W_self_1, W_neigh_1):
  adj4 = adj.reshape(N // PACK, 128)
  h0, h1, ns1 = _sc_gather_aggregate(features, adj4, batch1)
  return _tc_aggregate(h0, h1, ns1, W_self_0, W_neigh_0,
                       W_self_1, W_neigh_1)


# speedup vs baseline: 1.0236x; 1.0236x over previous
import functools

import jax
import jax.numpy as jnp
from jax import lax
from jax.experimental import pallas as pl
from jax.experimental.pallas import tpu as pltpu
from jax.experimental.pallas import tpu_sc as plsc

N, D, B, MAXDEG = 10000, 128, 1024, 32
S1, S2 = 25, 10
H = 128

NC, NS = 2, 16
NW = NC * NS
NB = B // NW
NSMP = NB * S2
GS = 80
NSTRIP = NSMP // GS
LANES = 16
PACK = 128 // MAXDEG


def _sc_gather_aggregate(features, adj4, batch1):
  mesh = plsc.VectorSubcoreMesh(core_axis_name="c", subcore_axis_name="s",
                                num_cores=NC, num_subcores=NS)

  @functools.partial(
      pl.kernel,
      out_type=[
          jax.ShapeDtypeStruct((B, D), jnp.float32),
          jax.ShapeDtypeStruct((B * S2, D), jnp.float32),
          jax.ShapeDtypeStruct((B * S2, D), jnp.float32),
      ],
      mesh=mesh,
      compiler_params=pltpu.CompilerParams(needs_layout_passes=False),
      scratch_types=[
          pltpu.VMEM((NB,), jnp.int32),
          pltpu.VMEM((NB, 128), jnp.int32),
          pltpu.VMEM((NSMP,), jnp.int32),
          pltpu.VMEM((NB, D), jnp.float32),
          [pltpu.VMEM((GS,), jnp.int32)] * 2,
          [pltpu.VMEM((GS,), jnp.int32)] * 2,
          [pltpu.VMEM((GS, 128), jnp.int32)] * 2,
          [pltpu.VMEM((GS, D), jnp.float32)] * 2,
          [pltpu.VMEM((GS, D), jnp.float32)] * 2,
          pltpu.VMEM((2 * GS, D), jnp.float32),
          pltpu.VMEM((2 * GS, D), jnp.float32),
          pltpu.VMEM((2 * GS,), jnp.int32),
          pltpu.VMEM((2 * GS,), jnp.int32),
          pltpu.SemaphoreType.DMA,
          pltpu.SemaphoreType.DMA,
          [pltpu.SemaphoreType.DMA] * 2,
          [pltpu.SemaphoreType.DMA] * 2,
          [pltpu.SemaphoreType.DMA] * 2,
          [pltpu.SemaphoreType.DMA] * 2,
      ],
  )
  def body(features_hbm, adj4_hbm, batch1_hbm, h0_out, h1_out, ns1_out,
           bidx_v, adj0_v, samp1_v, h0_v, sidx, soff, adj1, h1, ns1,
           colA_v, colB_v, idxA_v, idxB_v, semA, semB,
           semJ, semG, semH, semN):
    wid = lax.axis_index("c") * NS + lax.axis_index("s")
    base_b = wid * NB
    lane = lax.iota(jnp.int32, LANES)

    pltpu.sync_copy(batch1_hbm.at[pl.ds(base_b, NB)], bidx_v)

    pltpu.sync_copy(features_hbm.at[bidx_v], h0_v)
    pltpu.sync_copy(h0_v, h0_out.at[pl.ds(base_b, NB)])

    for j in range(NB // LANES):
      b = bidx_v[pl.ds(j * LANES, LANES)]
      bidx_v[pl.ds(j * LANES, LANES)] = b // PACK
    pltpu.sync_copy(adj4_hbm.at[bidx_v], adj0_v)

    pltpu.sync_copy(batch1_hbm.at[pl.ds(base_b, NB)], sidx[0].at[pl.ds(0, NB)])
    for j in range(NSMP // LANES):
      p = lane + j * LANES
      row = p // S2
      col = p - row * S2
      boff = (plsc.load_gather(sidx[0], [row]) % PACK) * MAXDEG
      v = plsc.load_gather(adj0_v, [row, boff + col])
      samp1_v[pl.ds(j * LANES, LANES)] = v

    def extract(l, idx_ref, bank, base=0):
      for j in range(GS // LANES):
        rows = lane + j * LANES
        off = soff[bank][pl.ds(j * LANES, LANES)]
        v = plsc.load_gather(adj1[bank], [rows, off + l])
        idx_ref[pl.ds(base + j * LANES, LANES)] = v

    def extract_pair(pp, idx_ref, bank):
      extract(2 * pp, idx_ref, bank, 0)
      extract(2 * pp + 1, idx_ref, bank, GS)

    def fire(idx_ref, col_ref, sem):
      pltpu.async_copy(features_hbm.at[idx_ref], col_ref, sem)

    def wait(idx_ref, col_ref, sem):
      pltpu.make_async_copy(features_hbm.at[idx_ref], col_ref, sem).wait()

    def acc_pair(col_ref, bank, first):
      @pl.loop(0, GS, unroll=2)
      def _a(r):
        for c in range(D // LANES):
          sl = pl.ds(c * LANES, LANES)
          if first:
            ns1[bank][r, sl] = col_ref[r, sl]
          else:
            plsc.addupdate(ns1[bank].at[r, sl], col_ref[r, sl])
          plsc.addupdate(ns1[bank].at[r, sl], col_ref[GS + r, sl])

    def acc_single(col_ref, bank):
      @pl.loop(0, GS, unroll=2)
      def _a(r):
        for c in range(D // LANES):
          sl = pl.ds(c * LANES, LANES)
          plsc.addupdate(ns1[bank].at[r, sl], col_ref[r, sl])

    def prep(t, bank):
      sbase = t * GS
      for j in range(GS // LANES):
        s = samp1_v[pl.ds(sbase + j * LANES, LANES)]
        soff[bank][pl.ds(j * LANES, LANES)] = (s % PACK) * MAXDEG
        sidx[bank][pl.ds(j * LANES, LANES)] = s // PACK

    def fire_strip(t, bank):
      pltpu.async_copy(adj4_hbm.at[sidx[bank]], adj1[bank], semJ[bank])
      pltpu.async_copy(
          features_hbm.at[samp1_v.at[pl.ds(t * GS, GS)]], h1[bank],
          semG[bank])

    def run_strip(t, bank, k, first_pair, last_pair):
      gbase = wid * NSMP + t * GS
      nbank = 1 - bank
      pltpu.make_async_copy(adj4_hbm.at[sidx[bank]], adj1[bank],
                            semJ[bank]).wait()

      if bank == 0:
        @pl.when(jnp.logical_not(first_pair))
        def _():
          pltpu.make_async_copy(h1[1], h1_out.at[pl.ds(0, GS)],
                                semH[1]).wait()
        prep(t + 1, nbank)
        fire_strip(t + 1, nbank)
      else:
        @pl.when(jnp.logical_not(last_pair))
        def _():
          pltpu.make_async_copy(h1[0], h1_out.at[pl.ds(0, GS)],
                                semH[0]).wait()
          prep(t + 1, nbank)
          fire_strip(t + 1, nbank)

      pltpu.make_async_copy(
          features_hbm.at[samp1_v.at[pl.ds(t * GS, GS)]], h1[bank],
          semG[bank]).wait()
      pltpu.async_copy(h1[bank], h1_out.at[pl.ds(gbase, GS)], semH[bank])

      @pl.when(jnp.logical_not(first_pair))
      def _():
        pltpu.make_async_copy(ns1[bank], ns1_out.at[pl.ds(0, GS)],
                              semN[bank]).wait()

      NP = S1 // 2
      extract_pair(0, idxA_v, bank)
      fire(idxA_v, colA_v, semA)
      extract_pair(1, idxB_v, bank)
      fire(idxB_v, colB_v, semB)
      wait(idxA_v, colA_v, semA)
      acc_pair(colA_v, bank, True)
      extract_pair(2, idxA_v, bank)
      fire(idxA_v, colA_v, semA)

      @pl.loop(0, (NP - 4) // 2)
      def _k(kk):
        wait(idxB_v, colB_v, semB)
        acc_pair(colB_v, bank, False)
        extract_pair(2 * kk + 3, idxB_v, bank)
        fire(idxB_v, colB_v, semB)
        wait(idxA_v, colA_v, semA)
        acc_pair(colA_v, bank, False)
        extract_pair(2 * kk + 4, idxA_v, bank)
        fire(idxA_v, colA_v, semA)

      wait(idxB_v, colB_v, semB)
      acc_pair(colB_v, bank, False)
      extract_pair(NP - 1, idxB_v, bank)
      fire(idxB_v, colB_v, semB)
      wait(idxA_v, colA_v, semA)
      acc_pair(colA_v, bank, False)
      extract(S1 - 1, idxA_v, bank, 0)
      pltpu.async_copy(features_hbm.at[idxA_v.at[pl.ds(0, GS)]],
                       colA_v.at[pl.ds(0, GS)], semA)
      wait(idxB_v, colB_v, semB)
      acc_pair(colB_v, bank, False)
      pltpu.make_async_copy(features_hbm.at[idxA_v.at[pl.ds(0, GS)]],
                            colA_v.at[pl.ds(0, GS)], semA).wait()
      acc_single(colA_v, bank)

      pltpu.async_copy(ns1[bank], ns1_out.at[pl.ds(gbase, GS)], semN[bank])

    prep(0, 0)
    fire_strip(0, 0)

    @pl.loop(0, NSTRIP // 2)
    def _pair(k):
      first = k == 0
      last = k == (NSTRIP // 2 - 1)
      run_strip(2 * k, 0, k, first, last)
      run_strip(2 * k + 1, 1, k, first, last)

    for bank in (0, 1):
      pltpu.make_async_copy(h1[bank], h1_out.at[pl.ds(0, GS)],
                            semH[bank]).wait()
      pltpu.make_async_copy(ns1[bank], ns1_out.at[pl.ds(0, GS)],
                            semN[bank]).wait()

  return body(features, adj4, batch1)


def _tc_body(h0_ref, h1_ref, ns1_ref, ws0_ref, wn0_ref, ws1_ref, wn1_ref,
             out_ref):
  f32 = jnp.float32
  h1 = h1_ref[...]
  ns1 = ns1_ref[...] / float(S1)
  ws0 = ws0_ref[...]
  wn0 = wn0_ref[...]
  hs1 = jnp.dot(h1, ws0, preferred_element_type=f32)
  hn1 = jnp.dot(ns1, wn0, preferred_element_type=f32)
  h1c = jnp.maximum(jnp.concatenate([hs1, hn1], axis=1), 0.0)

  bb = h0_ref.shape[0]
  ri = lax.broadcasted_iota(jnp.int32, (bb, bb * S2), 0)
  ci = lax.broadcasted_iota(jnp.int32, (bb, bb * S2), 1)
  pool = jnp.where(ci // S2 == ri, 1.0 / S2, 0.0).astype(f32)

  ns0 = jnp.dot(pool, h1, preferred_element_type=f32)
  h0 = h0_ref[...]
  h0c = jnp.maximum(
      jnp.concatenate([jnp.dot(h0, ws0, preferred_element_type=f32),
                       jnp.dot(ns0, wn0, preferred_element_type=f32)],
                      axis=1), 0.0)
  h1m = jnp.dot(pool, h1c, preferred_element_type=f32)
  out_ref[...] = jnp.concatenate(
      [jnp.dot(h0c, ws1_ref[...], preferred_element_type=f32),
       jnp.dot(h1m, wn1_ref[...], preferred_element_type=f32)], axis=1)


def _tc_aggregate(h0, h1, ns1, W_self_0, W_neigh_0, W_self_1, W_neigh_1):
  Bb = 128
  grid = (B // Bb,)
  return pl.pallas_call(
      _tc_body,
      out_shape=jax.ShapeDtypeStruct((B, 2 * H), jnp.float32),
      grid=grid,
      in_specs=[
          pl.BlockSpec((Bb, D), lambda i: (i, 0)),
          pl.BlockSpec((Bb * S2, D), lambda i: (i, 0)),
          pl.BlockSpec((Bb * S2, D), lambda i: (i, 0)),
          pl.BlockSpec((D, H), lambda i: (0, 0)),
          pl.BlockSpec((D, H), lambda i: (0, 0)),
          pl.BlockSpec((2 * H, H), lambda i: (0, 0)),
          pl.BlockSpec((2 * H, H), lambda i: (0, 0)),
      ],
      out_specs=pl.BlockSpec((Bb, 2 * H), lambda i: (i, 0)),
  )(h0, h1, ns1, W_self_0, W_neigh_0, W_self_1, W_neigh_1)


def kernel(features, adj, batch1, W_self_0, W_neigh_0, W_self_1, W_neigh_1):
  adj4 = adj.reshape(N // PACK, 128)
  h0, h1, ns1 = _sc_gather_aggregate(features, adj4, batch1)
  return _tc_aggregate(h0, h1, ns1, W_self_0, W_neigh_0,
                       W_self_1, W_neigh_1)
